# baseline (speedup 1.0000x reference)
import functools

import jax
import jax.numpy as jnp
from jax import lax
from jax.experimental import pallas as pl
from jax.experimental.pallas import tpu as pltpu
from jax.experimental.pallas import tpu_sc as plsc

VOCAB = 100000
EMB = 64
B = 4096
L = 200

NW = 32
BG = B // NW
NB = B // 128
NE = EMB // 8
K = 4
NIT = L // K


def _body(xt_hbm, emb_hbm, posx_hbm, out_hbm,
          idx0, idx1, rowsa0, rowsa1,
          rowsb0, rowsb1, rowsb2, rowsb3, posx0, posx1,
          sem_i0, sem_i1, sem_g0, sem_g1,
          sem_s0, sem_s1, sem_s2, sem_s3):
    idx_v = (idx0, idx1)
    rows_a = (rowsa0, rowsa1)
    rows_b = (rowsb0, rowsb1, rowsb2, rowsb3)
    posx_v = (posx0, posx1)
    sem_i = (sem_i0, sem_i1)
    sem_g = (sem_g0, sem_g1)
    sem_s = (sem_s0, sem_s1, sem_s2, sem_s3)

    sid = lax.axis_index("s")
    g = sid * 2 + lax.axis_index("c")

    def idx_load(m, b):
        pltpu.async_copy(
            xt_hbm.at[pl.ds(m * K, K), pl.ds(g * 128, 128)], idx_v[b], sem_i[b])

    def posx_load(m, b):
        pltpu.async_copy(posx_hbm.at[pl.ds(m * K, K)], posx_v[b], sem_i[b])

    def wait_idx(b):
        pltpu.make_async_copy(
            xt_hbm.at[pl.ds(0, K), pl.ds(0, 128)], idx_v[b], sem_i[b]).wait()
        pltpu.make_async_copy(posx_hbm.at[pl.ds(0, K)], posx_v[b], sem_i[b]).wait()

    def gather(b):
        for li in range(K):
            pltpu.async_copy(
                emb_hbm.at[idx_v[b].at[li]], rows_a[b].at[li], sem_g[b])

    def wait_gather(b):
        for li in range(K):
            pltpu.make_async_copy(
                emb_hbm.at[idx_v[b].at[li]], rows_a[b].at[li], sem_g[b]).wait()

    def store(l, b):
        pltpu.async_copy(
            rows_b[b].at[:, :, pl.ds(0, 128)], out_hbm.at[l, :, g], sem_s[b])

    def wait_store(b):
        pltpu.make_async_copy(
            rows_b[b].at[:, :, pl.ds(0, 128)], out_hbm.at[0, :, 0], sem_s[b]
        ).wait()

    for sb in range(K):
        pltpu.async_copy(
            out_hbm.at[0, :, g], rows_b[sb].at[:, :, pl.ds(0, 128)], sem_s[sb])

    idx_load(0, 0)
    posx_load(0, 0)
    idx_load(1, 1)
    posx_load(1, 1)
    wait_idx(0)
    gather(0)

    @pl.loop(0, NIT, step=2)
    def _pos(t):
        for db in range(2):
            m = t + db
            b = db
            o = 1 - db

            wait_gather(b)
            idx_load(lax.min(m + 2, NIT - 1), b)
            wait_idx(o)
            gather(o)

            lanes = lax.iota(jnp.int32, 16)
            d0c = [(lanes + e0 * 16) // 8 for e0 in range(EMB // 16)]
            d1c = [lax.rem(lanes + e0 * 16, 8) for e0 in range(EMB // 16)]
            for li in range(K):
                sb = li
                wait_store(sb)
                pvecs = [
                    posx_v[b][li, pl.ds(e0 * 16, 16)]
                    for e0 in range(EMB // 16)
                ]

                @plsc.parallel_loop(0, 128, unroll=8)
                def _tok(tok, li=li, sb=sb, pvecs=pvecs):
                    blv = lanes * 0 + tok
                    for e0 in range(EMB // 16):
                        v = rows_a[b][li, tok, pl.ds(e0 * 16, 16)]
                        plsc.store_scatter(
                            rows_b[sb], [d0c[e0], d1c[e0], blv], v + pvecs[e0])

                store(m * K + li, sb)
            posx_load(lax.min(m + 2, NIT - 1), b)

    wait_idx(1)
    wait_gather(0)
    for sb in range(K):
        wait_store(sb)


@jax.jit
def kernel(x, emb_table, pos_table):
    xt = x.T
    posx = pos_table[:L]

    mesh = plsc.VectorSubcoreMesh(core_axis_name="c", subcore_axis_name="s")
    out5 = pl.kernel(
        _body,
        out_type=jax.ShapeDtypeStruct((L, NE, NB, 8, 128), jnp.float32),
        mesh=mesh,
        compiler_params=pltpu.CompilerParams(use_tc_tiling_on_sc=False, needs_layout_passes=False),
        scratch_types=[
            pltpu.VMEM((K, 128), jnp.int32),
            pltpu.VMEM((K, 128), jnp.int32),
            pltpu.VMEM((K, 128, EMB), jnp.float32),
            pltpu.VMEM((K, 128, EMB), jnp.float32),
            pltpu.VMEM((NE, 8, 137), jnp.float32),
            pltpu.VMEM((NE, 8, 137), jnp.float32),
            pltpu.VMEM((NE, 8, 137), jnp.float32),
            pltpu.VMEM((NE, 8, 137), jnp.float32),
            pltpu.VMEM((K, EMB), jnp.float32),
            pltpu.VMEM((K, EMB), jnp.float32),
            pltpu.SemaphoreType.DMA,
            pltpu.SemaphoreType.DMA,
            pltpu.SemaphoreType.DMA,
            pltpu.SemaphoreType.DMA,
            pltpu.SemaphoreType.DMA,
            pltpu.SemaphoreType.DMA,
            pltpu.SemaphoreType.DMA,
            pltpu.SemaphoreType.DMA,
        ],
    )(xt, emb_table, posx)
    return out5.transpose(2, 4, 0, 1, 3).reshape(B, L, EMB)


# speedup vs baseline: 1.0061x; 1.0061x over previous
import functools

import jax
import jax.numpy as jnp
from jax import lax
from jax.experimental import pallas as pl
from jax.experimental.pallas import tpu as pltpu
from jax.experimental.pallas import tpu_sc as plsc

VOCAB = 100000
EMB = 64
B = 4096
L = 200

NW = 32
BG = B // NW
NB = B // 128
NE = EMB // 8
K = 4
NIT = L // K


def _body(xt_hbm, emb_hbm, posx_hbm, out_hbm,
          idx0, idx1, rowsa0, rowsa1, rowsb0, rowsb1, posx0, posx1,
          sem_i0, sem_i1, sem_g0, sem_g1, sem_s0, sem_s1):
    idx_v = (idx0, idx1)
    rows_a = (rowsa0, rowsa1)
    rows_b = (rowsb0, rowsb1)
    posx_v = (posx0, posx1)
    sem_i = (sem_i0, sem_i1)
    sem_g = (sem_g0, sem_g1)
    sem_s = (sem_s0, sem_s1)

    sid = lax.axis_index("s")
    g = sid * 2 + lax.axis_index("c")

    def idx_load(m, b):
        pltpu.async_copy(
            xt_hbm.at[pl.ds(m * K, K), pl.ds(g * 128, 128)], idx_v[b], sem_i[b])

    def posx_load(m, b):
        pltpu.async_copy(posx_hbm.at[pl.ds(m * K, K)], posx_v[b], sem_i[b])

    def wait_idx(b):
        pltpu.make_async_copy(
            xt_hbm.at[pl.ds(0, K), pl.ds(0, 128)], idx_v[b], sem_i[b]).wait()
        pltpu.make_async_copy(posx_hbm.at[pl.ds(0, K)], posx_v[b], sem_i[b]).wait()

    def gather(b):
        for li in range(K):
            pltpu.async_copy(
                emb_hbm.at[idx_v[b].at[li]], rows_a[b].at[li], sem_g[b])

    def wait_gather(b):
        for li in range(K):
            pltpu.make_async_copy(
                emb_hbm.at[idx_v[b].at[li]], rows_a[b].at[li], sem_g[b]).wait()

    def store(l, b):
        pltpu.async_copy(
            rows_b[b].at[:, :, pl.ds(0, 128)], out_hbm.at[l, :, g], sem_s[b])

    def wait_store(b):
        pltpu.make_async_copy(
            rows_b[b].at[:, :, pl.ds(0, 128)], out_hbm.at[0, :, 0], sem_s[b]
        ).wait()

    pltpu.async_copy(out_hbm.at[0, :, g], rows_b[0].at[:, :, pl.ds(0, 128)], sem_s[0])
    pltpu.async_copy(out_hbm.at[0, :, g], rows_b[1].at[:, :, pl.ds(0, 128)], sem_s[1])

    idx_load(0, 0)
    posx_load(0, 0)
    idx_load(1, 1)
    posx_load(1, 1)
    wait_idx(0)
    gather(0)

    @pl.loop(0, NIT, step=2)
    def _pos(t):
        for db in range(2):
            m = t + db
            b = db
            o = 1 - db

            wait_gather(b)
            idx_load(lax.min(m + 2, NIT - 1), b)
            wait_idx(o)
            gather(o)

            lanes = lax.iota(jnp.int32, 16)
            d0c = [(lanes + e0 * 16) // 8 for e0 in range(EMB // 16)]
            d1c = [lax.rem(lanes + e0 * 16, 8) for e0 in range(EMB // 16)]
            for li in range(K):
                sb = li % 2
                wait_store(sb)
                pvecs = [
                    posx_v[b][li, pl.ds(e0 * 16, 16)]
                    for e0 in range(EMB // 16)
                ]

                @plsc.parallel_loop(0, 128, unroll=8)
                def _tok(tok, li=li, sb=sb, pvecs=pvecs):
                    blv = lanes * 0 + tok
                    for e0 in range(EMB // 16):
                        v = rows_a[b][li, tok, pl.ds(e0 * 16, 16)]
                        plsc.store_scatter(
                            rows_b[sb], [d0c[e0], d1c[e0], blv], v + pvecs[e0])

                store(m * K + li, sb)
            posx_load(lax.min(m + 2, NIT - 1), b)

    wait_idx(1)
    wait_gather(0)
    wait_store(0)
    wait_store(1)


@jax.jit
def kernel(x, emb_table, pos_table):
    xt = x.T
    posx = pos_table[:L]

    mesh = plsc.VectorSubcoreMesh(core_axis_name="c", subcore_axis_name="s")
    out5 = pl.kernel(
        _body,
        out_type=jax.ShapeDtypeStruct((L, NE, NB, 8, 128), jnp.float32),
        mesh=mesh,
        compiler_params=pltpu.CompilerParams(use_tc_tiling_on_sc=False, needs_layout_passes=False),
        scratch_types=[
            pltpu.VMEM((K, 128), jnp.int32),
            pltpu.VMEM((K, 128), jnp.int32),
            pltpu.VMEM((K, 128, EMB), jnp.float32),
            pltpu.VMEM((K, 128, EMB), jnp.float32),
            pltpu.VMEM((NE, 8, 137), jnp.float32),
            pltpu.VMEM((NE, 8, 137), jnp.float32),
            pltpu.VMEM((K, EMB), jnp.float32),
            pltpu.VMEM((K, EMB), jnp.float32),
            pltpu.SemaphoreType.DMA,
            pltpu.SemaphoreType.DMA,
            pltpu.SemaphoreType.DMA,
            pltpu.SemaphoreType.DMA,
            pltpu.SemaphoreType.DMA,
            pltpu.SemaphoreType.DMA,
        ],
    )(xt, emb_table, posx)
    return out5.transpose(2, 4, 0, 1, 3).reshape(B, L, EMB)
